# baseline (speedup 1.0000x reference)
import jax
import jax.numpy as jnp
from jax import lax
from jax.experimental import pallas as pl
from jax.experimental.pallas import tpu as pltpu
from jax.experimental.pallas import tpu_sc as plsc

DIM = 64
EPS = 1e-5
NC, NS = 2, 16
NW = NC * NS
RPC = 4
NQ = DIM // 16


def _rsqrt_newton(x):
    i = lax.bitcast_convert_type(x, jnp.int32)
    i = jnp.int32(0x5F3759DF) - (i >> 1)
    y = lax.bitcast_convert_type(i, jnp.float32)
    for _ in range(3):
        y = y * (1.5 - 0.5 * x * y * y)
    return y


def _fused_sc(i0, i1, t0, t1, gamma, beta):
    batch, fields = i0.shape
    rows_w = batch // NW
    chunks = rows_w // RPC
    assert chunks % 2 == 0
    mesh = plsc.VectorSubcoreMesh(core_axis_name="c", subcore_axis_name="s")

    def body(i0_hbm, i1_hbm, t0_hbm, t1_hbm, g_hbm, b_hbm, out_hbm,
             i0_v, i1_v, gb_v,
             r0a, r1a, r0b, r1b, oa, ob,
             sga, sgb, soa, sob):
        wid = lax.axis_index("s") * NC + lax.axis_index("c")
        row0 = wid * rows_w

        pltpu.sync_copy(i0_hbm.at[pl.ds(row0, rows_w)], i0_v)
        pltpu.sync_copy(i1_hbm.at[pl.ds(row0, rows_w)], i1_v)
        pltpu.sync_copy(g_hbm, gb_v.at[0])
        pltpu.sync_copy(b_hbm, gb_v.at[1])
        gv = [gb_v[0, pl.ds(16 * q, 16)] for q in range(NQ)]
        bv = [gb_v[1, pl.ds(16 * q, 16)] for q in range(NQ)]

        def issue_gathers(j, r0x, r1x, sgx):
            for i in range(RPC):
                pltpu.async_copy(t0_hbm.at[i0_v.at[j * RPC + i]], r0x.at[i], sgx)
                pltpu.async_copy(t1_hbm.at[i1_v.at[j * RPC + i]], r1x.at[i], sgx)

        def wait_gathers(r0x, r1x, sgx):
            for i in range(RPC):
                pltpu.make_async_copy(t0_hbm.at[i0_v.at[0]], r0x.at[i], sgx).wait()
                pltpu.make_async_copy(t1_hbm.at[i1_v.at[0]], r1x.at[i], sgx).wait()

        def out_dst(j):
            return out_hbm.at[pl.ds(row0 + j * RPC, RPC)]

        def compute(r0x, r1x, ox):
            @plsc.parallel_loop(0, RPC * fields, 1, unroll=8)
            def row(k):
                i = k // fields
                j = k - i * fields
                a = [r0x[i, j, pl.ds(16 * q, 16)] + r1x[i, j, pl.ds(16 * q, 16)]
                     for q in range(NQ)]
                tot = jnp.sum((a[0] + a[1]) + (a[2] + a[3]))
                tot2 = jnp.sum((a[0] * a[0] + a[1] * a[1])
                               + (a[2] * a[2] + a[3] * a[3]))
                mu = tot * (1.0 / DIM)
                var = tot2 * (1.0 / DIM) - mu * mu
                rstd = _rsqrt_newton(var + EPS)
                for q in range(NQ):
                    ox[i, j, pl.ds(16 * q, 16)] = \
                        (a[q] - mu) * (rstd * gv[q]) + bv[q]

        issue_gathers(0, r0a, r1a, sga)
        pltpu.async_copy(oa, out_dst(0), soa)
        pltpu.async_copy(ob, out_dst(1), sob)

        def pair(p, carry):
            ja = 2 * p
            wait_gathers(r0a, r1a, sga)
            issue_gathers(ja + 1, r0b, r1b, sgb)
            pltpu.make_async_copy(oa, out_dst(0), soa).wait()
            compute(r0a, r1a, oa)
            pltpu.async_copy(oa, out_dst(ja), soa)
            wait_gathers(r0b, r1b, sgb)

            @pl.when(p < chunks // 2 - 1)
            def _():
                issue_gathers(ja + 2, r0a, r1a, sga)

            pltpu.make_async_copy(ob, out_dst(0), sob).wait()
            compute(r0b, r1b, ob)
            pltpu.async_copy(ob, out_dst(ja + 1), sob)
            return carry

        lax.fori_loop(0, chunks // 2, pair, 0)
        pltpu.make_async_copy(oa, out_dst(0), soa).wait()
        pltpu.make_async_copy(ob, out_dst(0), sob).wait()

    f = pl.kernel(
        body,
        out_type=jax.ShapeDtypeStruct((batch, fields, DIM), jnp.float32),
        mesh=mesh,
        scratch_types=[
            pltpu.VMEM((rows_w, fields), jnp.int32),
            pltpu.VMEM((rows_w, fields), jnp.int32),
            pltpu.VMEM((2, DIM), jnp.float32),
            pltpu.VMEM((RPC, fields, DIM), jnp.float32),
            pltpu.VMEM((RPC, fields, DIM), jnp.float32),
            pltpu.VMEM((RPC, fields, DIM), jnp.float32),
            pltpu.VMEM((RPC, fields, DIM), jnp.float32),
            pltpu.VMEM((RPC, fields, DIM), jnp.float32),
            pltpu.VMEM((RPC, fields, DIM), jnp.float32),
            pltpu.SemaphoreType.DMA,
            pltpu.SemaphoreType.DMA,
            pltpu.SemaphoreType.DMA,
            pltpu.SemaphoreType.DMA,
        ],
        compiler_params=pltpu.CompilerParams(
            use_tc_tiling_on_sc=False, needs_layout_passes=False),
    )
    return f(i0, i1, t0, t1, gamma, beta)


def kernel(idx0, idx1, table0, table1, gamma, beta):
    return _fused_sc(idx0.astype(jnp.int32), idx1.astype(jnp.int32),
                     table0, table1, gamma, beta)


# speedup vs baseline: 1.0198x; 1.0198x over previous
import jax
import jax.numpy as jnp
from jax import lax
from jax.experimental import pallas as pl
from jax.experimental.pallas import tpu as pltpu
from jax.experimental.pallas import tpu_sc as plsc

DIM = 64
EPS = 1e-5
NC, NS = 2, 16
NW = NC * NS
RPC = 4
NQ = DIM // 16


def _rsqrt_newton(x):
    i = lax.bitcast_convert_type(x, jnp.int32)
    i = jnp.int32(0x5F3759DF) - (i >> 1)
    y = lax.bitcast_convert_type(i, jnp.float32)
    for _ in range(3):
        y = y * (1.5 - 0.5 * x * y * y)
    return y


def _fused_sc(i0, i1, ct, gamma, beta):
    batch, fields = i0.shape
    rows_w = batch // NW
    chunks = rows_w // RPC
    assert chunks % 2 == 0
    mesh = plsc.VectorSubcoreMesh(core_axis_name="c", subcore_axis_name="s")

    def body(i0_hbm, i1_hbm, ct_hbm, g_hbm, b_hbm, out_hbm,
             i0_v, i1_v, gb_v,
             r0a, r1a, r0b, r1b, oa, ob,
             sga, sgb, soa, sob):
        wid = lax.axis_index("s") * NC + lax.axis_index("c")
        row0 = wid * rows_w

        pltpu.sync_copy(i0_hbm.at[pl.ds(row0, rows_w)], i0_v)
        pltpu.sync_copy(i1_hbm.at[pl.ds(row0, rows_w)], i1_v)
        pltpu.sync_copy(g_hbm, gb_v.at[0])
        pltpu.sync_copy(b_hbm, gb_v.at[1])
        gv = [gb_v[0, pl.ds(16 * q, 16)] for q in range(NQ)]
        bv = [gb_v[1, pl.ds(16 * q, 16)] for q in range(NQ)]

        def issue_gathers(j, r0x, r1x, sgx):
            for i in range(RPC):
                pltpu.async_copy(ct_hbm.at[i0_v.at[j * RPC + i]], r0x.at[i], sgx)
                pltpu.async_copy(ct_hbm.at[i1_v.at[j * RPC + i]], r1x.at[i], sgx)

        def wait_gathers(r0x, r1x, sgx):
            for i in range(RPC):
                pltpu.make_async_copy(ct_hbm.at[i0_v.at[0]], r0x.at[i], sgx).wait()
                pltpu.make_async_copy(ct_hbm.at[i1_v.at[0]], r1x.at[i], sgx).wait()

        def out_dst(j):
            return out_hbm.at[pl.ds(row0 + j * RPC, RPC)]

        def compute(r0x, r1x, ox):
            @plsc.parallel_loop(0, RPC * fields, 1, unroll=8)
            def row(k):
                i = k // fields
                j = k - i * fields
                a = [r0x[i, j, pl.ds(16 * q, 16)]
                     + r1x[i, j, pl.ds(DIM + 16 * q, 16)]
                     for q in range(NQ)]
                tot = jnp.sum((a[0] + a[1]) + (a[2] + a[3]))
                tot2 = jnp.sum((a[0] * a[0] + a[1] * a[1])
                               + (a[2] * a[2] + a[3] * a[3]))
                mu = tot * (1.0 / DIM)
                var = tot2 * (1.0 / DIM) - mu * mu
                rstd = _rsqrt_newton(var + EPS)
                for q in range(NQ):
                    ox[i, j, pl.ds(16 * q, 16)] = \
                        (a[q] - mu) * (rstd * gv[q]) + bv[q]

        issue_gathers(0, r0a, r1a, sga)
        pltpu.async_copy(oa, out_dst(0), soa)
        pltpu.async_copy(ob, out_dst(1), sob)

        def pair(p, carry):
            ja = 2 * p
            wait_gathers(r0a, r1a, sga)
            issue_gathers(ja + 1, r0b, r1b, sgb)
            pltpu.make_async_copy(oa, out_dst(0), soa).wait()
            compute(r0a, r1a, oa)
            pltpu.async_copy(oa, out_dst(ja), soa)
            wait_gathers(r0b, r1b, sgb)

            @pl.when(p < chunks // 2 - 1)
            def _():
                issue_gathers(ja + 2, r0a, r1a, sga)

            pltpu.make_async_copy(ob, out_dst(0), sob).wait()
            compute(r0b, r1b, ob)
            pltpu.async_copy(ob, out_dst(ja + 1), sob)
            return carry

        lax.fori_loop(0, chunks // 2, pair, 0)
        pltpu.make_async_copy(oa, out_dst(0), soa).wait()
        pltpu.make_async_copy(ob, out_dst(0), sob).wait()

    f = pl.kernel(
        body,
        out_type=jax.ShapeDtypeStruct((batch, fields, DIM), jnp.float32),
        mesh=mesh,
        scratch_types=[
            pltpu.VMEM((rows_w, fields), jnp.int32),
            pltpu.VMEM((rows_w, fields), jnp.int32),
            pltpu.VMEM((2, DIM), jnp.float32),
            pltpu.VMEM((RPC, fields, 2 * DIM), jnp.float32),
            pltpu.VMEM((RPC, fields, 2 * DIM), jnp.float32),
            pltpu.VMEM((RPC, fields, 2 * DIM), jnp.float32),
            pltpu.VMEM((RPC, fields, 2 * DIM), jnp.float32),
            pltpu.VMEM((RPC, fields, DIM), jnp.float32),
            pltpu.VMEM((RPC, fields, DIM), jnp.float32),
            pltpu.SemaphoreType.DMA,
            pltpu.SemaphoreType.DMA,
            pltpu.SemaphoreType.DMA,
            pltpu.SemaphoreType.DMA,
        ],
        compiler_params=pltpu.CompilerParams(
            use_tc_tiling_on_sc=False, needs_layout_passes=False),
    )
    return f(i0, i1, ct, gamma, beta)


def kernel(idx0, idx1, table0, table1, gamma, beta):
    ct = jnp.concatenate([table0, table1], axis=1)
    return _fused_sc(idx0.astype(jnp.int32), idx1.astype(jnp.int32),
                     ct, gamma, beta)
